# Initial kernel scaffold; baseline (speedup 1.0000x reference)
import functools
import math

import jax
import jax.numpy as jnp
from jax import lax
from jax.experimental import pallas as pl
from jax.experimental.pallas import tpu as pltpu
from jax.experimental.pallas import tpu_sc as plsc

N = 10000
NPAD = 10240
E = 320000
D = 128
CH = 128
NCHUNK = E // CH
NW = 32
CPT = (NCHUNK + NW - 1) // NW
ROWS_PER_TILE = NPAD // 16
_INV_SQRT_D = 1.0 / math.sqrt(D)


def _edge_body(q_hbm, k_hbm, v_hbm, src_hbm, dst_hbm,
               acc_out, den_out,
               srci, dsti, qd, kk, vv, exb,
               accs, dens,
               sq, sk, sv):
    cid = lax.axis_index("c")
    sid = lax.axis_index("s")
    wid = sid * 2 + cid
    z16 = jnp.zeros((16,), jnp.float32)

    def _z(i, c):
        exb[i] = z16
        for j in range(8):
            qd[i, pl.ds(16 * j, 16)] = z16
        return c
    lax.fori_loop(0, CH, _z, 0)
    r0 = sid * ROWS_PER_TILE
    for i in range(ROWS_PER_TILE // CH):
        pltpu.sync_copy(qd, accs.at[pl.ds(r0 + i * CH, CH)])
        pltpu.sync_copy(exb, dens.at[pl.ds(r0 + i * CH, CH)])
    plsc.subcore_barrier()

    inv = jnp.float32(_INV_SQRT_D)

    def _chunk(it, c):
        chunk = it * NW + wid

        @pl.when(chunk < NCHUNK)
        def _():
            base = chunk * CH
            pltpu.sync_copy(dst_hbm.at[pl.ds(base, CH)], dsti)
            pltpu.sync_copy(src_hbm.at[pl.ds(base, CH)], srci)
            cq = pltpu.async_copy(q_hbm.at[dsti], qd, sq)
            ck = pltpu.async_copy(k_hbm.at[srci], kk, sk)
            cv = pltpu.async_copy(v_hbm.at[srci], vv, sv)
            cq.wait()
            ck.wait()
            cv.wait()

            def _edge(e, c2):
                a = qd[e, pl.ds(0, 16)] * kk[e, pl.ds(0, 16)]
                for j in range(1, 8):
                    a = a + qd[e, pl.ds(16 * j, 16)] * kk[e, pl.ds(16 * j, 16)]
                s = jnp.sum(a) * inv
                ev = jnp.exp(jnp.broadcast_to(s, (16,)))
                exb[e] = ev
                for j in range(8):
                    sl = pl.ds(16 * j, 16)
                    vv[e, sl] = vv[e, sl] * ev
                return c2
            lax.fori_loop(0, CH, _edge, 0)

            pltpu.sync_copy(vv, accs.at[dsti], add=True)
            pltpu.sync_copy(exb, dens.at[dsti], add=True)
        return c
    lax.fori_loop(0, CPT, _chunk, 0)

    plsc.subcore_barrier()
    pltpu.sync_copy(accs.at[pl.ds(r0, ROWS_PER_TILE)],
                    acc_out.at[cid, pl.ds(r0, ROWS_PER_TILE)])
    pltpu.sync_copy(dens.at[pl.ds(r0, ROWS_PER_TILE)],
                    den_out.at[cid, pl.ds(r0, ROWS_PER_TILE)])


_edge_call = pl.kernel(
    _edge_body,
    out_type=[
        jax.ShapeDtypeStruct((2, NPAD, D), jnp.float32),
        jax.ShapeDtypeStruct((2, NPAD, 16), jnp.float32),
    ],
    mesh=plsc.VectorSubcoreMesh(core_axis_name="c", subcore_axis_name="s"),
    scratch_types=[
        pltpu.VMEM((CH,), jnp.int32),
        pltpu.VMEM((CH,), jnp.int32),
        pltpu.VMEM((CH, D), jnp.float32),
        pltpu.VMEM((CH, D), jnp.float32),
        pltpu.VMEM((CH, D), jnp.float32),
        pltpu.VMEM((CH, 16), jnp.float32),
        pltpu.VMEM_SHARED((NPAD, D), jnp.float32),
        pltpu.VMEM_SHARED((NPAD, 16), jnp.float32),
        pltpu.SemaphoreType.DMA,
        pltpu.SemaphoreType.DMA,
        pltpu.SemaphoreType.DMA,
    ],
)


def _mm(a, w):
    return lax.dot_general(a, w, (((1,), (1,)), ((), ())),
                           preferred_element_type=jnp.float32)


def _proj3_body(x_ref, qw, qb, kw, kb, vw, vb, q_o, k_o, v_o):
    xv = x_ref[...]
    q_o[...] = _mm(xv, qw[...]) + qb[...]
    k_o[...] = _mm(xv, kw[...]) + kb[...]
    v_o[...] = _mm(xv, vw[...]) + vb[...]


_proj3 = pl.pallas_call(
    _proj3_body,
    out_shape=[jax.ShapeDtypeStruct((N, D), jnp.float32)] * 3,
)


def _combine(acc_r, den_r, skip):
    acc = acc_r[0, :N, :] + acc_r[1, :N, :]
    den = den_r[0, :N, 0:1] + den_r[1, :N, 0:1]
    return acc / (den + 1e-16) + skip


def _bn_relu(h, g, b):
    r = jnp.maximum(h, 0.0)
    mu = jnp.mean(r, axis=0, keepdims=True)
    var = jnp.mean((r - mu) ** 2, axis=0, keepdims=True)
    return (r - mu) * lax.rsqrt(var + 1e-5) * g + b


def _mid_body(acc_r, den_r, x_ref, sw, sb, g1, b1,
              qw2, qb2, kw2, kb2, vw2, vb2,
              h_o, q_o, k_o, v_o):
    h = _combine(acc_r, den_r, _mm(x_ref[...], sw[...]) + sb[...])
    hn = _bn_relu(h, g1[...], b1[...])
    h_o[...] = hn
    q_o[...] = _mm(hn, qw2[...]) + qb2[...]
    k_o[...] = _mm(hn, kw2[...]) + kb2[...]
    v_o[...] = _mm(hn, vw2[...]) + vb2[...]


_mid = pl.pallas_call(
    _mid_body,
    out_shape=[jax.ShapeDtypeStruct((N, D), jnp.float32)] * 4,
)


def _fin_body(acc_r, den_r, h_ref, sw, sb, g2, b2, batch_ref, pw, pb,
              out_o, h2n_ref, pooled_ref):
    h = _combine(acc_r, den_r, _mm(h_ref[...], sw[...]) + sb[...])
    h2n_ref[...] = _bn_relu(h, g2[...], b2[...])

    def _g(g, c):
        m = jnp.max(jnp.where(batch_ref[...] == g, h2n_ref[...], -jnp.inf),
                    axis=0, keepdims=True)
        pooled_ref[pl.ds(g, 1), :] = m
        return c
    lax.fori_loop(0, 64, _g, 0)
    pooled = pooled_ref[...]
    pooled = jnp.where(jnp.isfinite(pooled), pooled, 0.0)
    out_o[...] = _mm(pooled, pw[...]) + pb[...]


_fin = pl.pallas_call(
    _fin_body,
    out_shape=jax.ShapeDtypeStruct((64, 1), jnp.float32),
    scratch_shapes=[
        pltpu.VMEM((N, D), jnp.float32),
        pltpu.VMEM((64, D), jnp.float32),
    ],
)


def kernel(x, edge_index, batch,
           c1_qw, c1_qb, c1_kw, c1_kb, c1_vw, c1_vb, c1_sw, c1_sb,
           c2_qw, c2_qb, c2_kw, c2_kb, c2_vw, c2_vb, c2_sw, c2_sb,
           bn1_g, bn1_b, bn2_g, bn2_b, pw, pb):
    src = edge_index[0]
    dst = edge_index[1]
    b2d = batch.reshape(N, 1)
    row = lambda a: a.reshape(1, -1)

    q1, k1, v1 = _proj3(x, c1_qw, row(c1_qb), c1_kw, row(c1_kb),
                        c1_vw, row(c1_vb))
    acc1, den1 = _edge_call(q1, k1, v1, src, dst)
    h1n, q2, k2, v2 = _mid(acc1, den1, x, c1_sw, row(c1_sb),
                           row(bn1_g), row(bn1_b),
                           c2_qw, row(c2_qb), c2_kw, row(c2_kb),
                           c2_vw, row(c2_vb))
    acc2, den2 = _edge_call(q2, k2, v2, src, dst)
    return _fin(acc2, den2, h1n, c2_sw, row(c2_sb),
                row(bn2_g), row(bn2_b), b2d, pw, row(pb))


# speedup vs baseline: 7.0257x; 7.0257x over previous
import functools
import math

import jax
import jax.numpy as jnp
from jax import lax
from jax.experimental import pallas as pl
from jax.experimental.pallas import tpu as pltpu
from jax.experimental.pallas import tpu_sc as plsc

N = 10000
NPAD = 10112
E = 320000
D = 128
CH = 80
NCHUNK = E // CH
NW = 32
CPT = (NCHUNK + NW - 1) // NW
ROWS_PER_TILE = NPAD // 16
_INV_SQRT_D = 1.0 / math.sqrt(D)


def _edge_body(q_hbm, k_hbm, v_hbm, src_hbm, dst_hbm,
               acc_out, den_out,
               srci, dsti, qd, kk, exb,
               accs, dens,
               sq, sk):
    cid = lax.axis_index("c")
    sid = lax.axis_index("s")
    wid = sid * 2 + cid
    z16 = jnp.zeros((16,), jnp.float32)

    def _z(i, c):
        exb[i] = z16
        for j in range(8):
            qd[i, pl.ds(16 * j, 16)] = z16
        return c
    lax.fori_loop(0, CH, _z, 0)
    r0 = sid * ROWS_PER_TILE
    for i in range(ROWS_PER_TILE // CH):
        pltpu.sync_copy(qd, accs.at[pl.ds(r0 + i * CH, CH)])
        pltpu.sync_copy(exb, dens.at[pl.ds(r0 + i * CH, CH)])
    _tail = ROWS_PER_TILE % CH
    _tb = r0 + (ROWS_PER_TILE // CH) * CH
    if _tail:
        pltpu.sync_copy(qd.at[pl.ds(0, _tail)], accs.at[pl.ds(_tb, _tail)])
        pltpu.sync_copy(exb.at[pl.ds(0, _tail)], dens.at[pl.ds(_tb, _tail)])
    plsc.subcore_barrier()
    for i in range(ROWS_PER_TILE // CH):
        pltpu.sync_copy(qd, acc_out.at[cid, pl.ds(r0 + i * CH, CH)])
        pltpu.sync_copy(exb, den_out.at[cid, pl.ds(r0 + i * CH, CH)])
    if _tail:
        pltpu.sync_copy(qd.at[pl.ds(0, _tail)], acc_out.at[cid, pl.ds(_tb, _tail)])
        pltpu.sync_copy(exb.at[pl.ds(0, _tail)], den_out.at[cid, pl.ds(_tb, _tail)])

    inv = jnp.float32(_INV_SQRT_D)

    def _chunk(it, c):
        chunk = it * NW + wid

        @pl.when(chunk < NCHUNK)
        def _():
            base = chunk * CH
            pltpu.sync_copy(dst_hbm.at[pl.ds(base, CH)], dsti)
            pltpu.sync_copy(src_hbm.at[pl.ds(base, CH)], srci)
            cq = pltpu.async_copy(q_hbm.at[dsti], qd, sq)
            ck = pltpu.async_copy(k_hbm.at[srci], kk, sk)
            cq.wait()
            ck.wait()

            def _edge(e, c2):
                a = qd[e, pl.ds(0, 16)] * kk[e, pl.ds(0, 16)]
                for j in range(1, 8):
                    a = a + qd[e, pl.ds(16 * j, 16)] * kk[e, pl.ds(16 * j, 16)]
                s = jnp.sum(a) * inv
                exb[e] = jnp.exp(jnp.broadcast_to(s, (16,)))
                return c2
            lax.fori_loop(0, CH, _edge, 0)

            cv = pltpu.async_copy(v_hbm.at[srci], qd, sq)
            cv.wait()

            def _scale(e, c2):
                ev = exb[e]
                for j in range(8):
                    sl = pl.ds(16 * j, 16)
                    qd[e, sl] = qd[e, sl] * ev
                return c2
            lax.fori_loop(0, CH, _scale, 0)

            pltpu.sync_copy(qd, accs.at[dsti], add=True)
            pltpu.sync_copy(exb, dens.at[dsti], add=True)
        return c
    lax.fori_loop(0, CPT, _chunk, 0)

    plsc.subcore_barrier()
    pltpu.sync_copy(accs.at[pl.ds(r0, ROWS_PER_TILE)],
                    acc_out.at[cid, pl.ds(r0, ROWS_PER_TILE)])
    pltpu.sync_copy(dens.at[pl.ds(r0, ROWS_PER_TILE)],
                    den_out.at[cid, pl.ds(r0, ROWS_PER_TILE)])


_edge_call = pl.kernel(
    _edge_body,
    out_type=[
        jax.ShapeDtypeStruct((2, NPAD, D), jnp.float32),
        jax.ShapeDtypeStruct((2, NPAD, 16), jnp.float32),
    ],
    mesh=plsc.VectorSubcoreMesh(core_axis_name="c", subcore_axis_name="s"),
    compiler_params=pltpu.CompilerParams(needs_layout_passes=False,
                                         use_tc_tiling_on_sc=False),
    scratch_types=[
        pltpu.VMEM((CH,), jnp.int32),
        pltpu.VMEM((CH,), jnp.int32),
        pltpu.VMEM((CH, D), jnp.float32),
        pltpu.VMEM((CH, D), jnp.float32),
        pltpu.VMEM((CH, 16), jnp.float32),
        pltpu.VMEM_SHARED((NPAD, D), jnp.float32),
        pltpu.VMEM_SHARED((NPAD, 16), jnp.float32),
        pltpu.SemaphoreType.DMA,
        pltpu.SemaphoreType.DMA,
    ],
)


def _mm(a, w):
    return lax.dot_general(a, w, (((1,), (1,)), ((), ())),
                           preferred_element_type=jnp.float32)


def _proj3_body(x_ref, qw, qb, kw, kb, vw, vb, q_o, k_o, v_o):
    xv = x_ref[...]
    q_o[...] = _mm(xv, qw[...]) + qb[...]
    k_o[...] = _mm(xv, kw[...]) + kb[...]
    v_o[...] = _mm(xv, vw[...]) + vb[...]


_proj3 = pl.pallas_call(
    _proj3_body,
    out_shape=[jax.ShapeDtypeStruct((N, D), jnp.float32)] * 3,
)


def _combine(acc_r, den_r, skip):
    acc = acc_r[0, :N, :] + acc_r[1, :N, :]
    den = den_r[0, :N, 0:1] + den_r[1, :N, 0:1]
    return acc / (den + 1e-16) + skip


def _bn_relu(h, g, b):
    r = jnp.maximum(h, 0.0)
    mu = jnp.mean(r, axis=0, keepdims=True)
    var = jnp.mean((r - mu) ** 2, axis=0, keepdims=True)
    return (r - mu) * lax.rsqrt(var + 1e-5) * g + b


def _mid_body(acc_r, den_r, x_ref, sw, sb, g1, b1,
              qw2, qb2, kw2, kb2, vw2, vb2,
              h_o, q_o, k_o, v_o):
    h = _combine(acc_r, den_r, _mm(x_ref[...], sw[...]) + sb[...])
    hn = _bn_relu(h, g1[...], b1[...])
    h_o[...] = hn
    q_o[...] = _mm(hn, qw2[...]) + qb2[...]
    k_o[...] = _mm(hn, kw2[...]) + kb2[...]
    v_o[...] = _mm(hn, vw2[...]) + vb2[...]


_mid = pl.pallas_call(
    _mid_body,
    out_shape=[jax.ShapeDtypeStruct((N, D), jnp.float32)] * 4,
)


def _fin_body(acc_r, den_r, h_ref, sw, sb, g2, b2, batch_ref, pw, pb,
              out_o, h2n_ref, pooled_ref):
    h = _combine(acc_r, den_r, _mm(h_ref[...], sw[...]) + sb[...])
    h2n_ref[...] = _bn_relu(h, g2[...], b2[...])

    def _g(g, c):
        m = jnp.max(jnp.where(batch_ref[...] == g, h2n_ref[...], -jnp.inf),
                    axis=0, keepdims=True)
        pooled_ref[pl.ds(g, 1), :] = m
        return c
    lax.fori_loop(0, 64, _g, 0)
    pooled = pooled_ref[...]
    pooled = jnp.where(jnp.isfinite(pooled), pooled, 0.0)
    out_o[...] = jnp.sum(pooled * pw[...], axis=1, keepdims=True) + pb[...]


_fin = pl.pallas_call(
    _fin_body,
    out_shape=jax.ShapeDtypeStruct((64, 1), jnp.float32),
    scratch_shapes=[
        pltpu.VMEM((N, D), jnp.float32),
        pltpu.VMEM((64, D), jnp.float32),
    ],
)


def kernel(x, edge_index, batch,
           c1_qw, c1_qb, c1_kw, c1_kb, c1_vw, c1_vb, c1_sw, c1_sb,
           c2_qw, c2_qb, c2_kw, c2_kb, c2_vw, c2_vb, c2_sw, c2_sb,
           bn1_g, bn1_b, bn2_g, bn2_b, pw, pb):
    src = edge_index[0]
    dst = edge_index[1]
    b2d = batch.reshape(N, 1)
    row = lambda a: a.reshape(1, -1)

    q1, k1, v1 = _proj3(x, c1_qw, row(c1_qb), c1_kw, row(c1_kb),
                        c1_vw, row(c1_vb))
    acc1, den1 = _edge_call(q1, k1, v1, src, dst)
    h1n, q2, k2, v2 = _mid(acc1, den1, x, c1_sw, row(c1_sb),
                           row(bn1_g), row(bn1_b),
                           c2_qw, row(c2_qb), c2_kw, row(c2_kb),
                           c2_vw, row(c2_vb))
    acc2, den2 = _edge_call(q2, k2, v2, src, dst)
    return _fin(acc2, den2, h1n, c2_sw, row(c2_sb),
                row(bn2_g), row(bn2_b), b2d, pw, row(pb))
